# Initial kernel scaffold; baseline (speedup 1.0000x reference)
import jax
import jax.numpy as jnp
from jax.experimental import pallas as pl


def kernel(x, edge_index, edge_attr, W_msg0, W_edge0, W_self0, W_agg0, W_msg1, W_edge1, W_self1, W_agg1, W_out):
    raise NotImplementedError("write your pallas kernel here")


# speedup vs baseline: 1.7618x; 1.7618x over previous
import jax
import jax.numpy as jnp
from jax import lax
from jax.experimental import pallas as pl
from jax.experimental.pallas import tpu as pltpu
from jax.experimental.pallas import tpu_sc as plsc

N = 10000
E = 160000
D = 256
DH = 128
NC = 2
NS = 16
EB = 80
EPT = E // NS
STEPS = EPT // EB
ROWS_PT = 624
TAIL0 = ROWS_PT * NS
TAILR = N - TAIL0
EB_D = 40
EPT_D = E // (NC * NS)
STEPS_D = EPT_D // EB_D
ZR = 48

_f32 = jnp.float32


def _mm_split_body(x_ref, w_ref, o_ref):
    r = jnp.dot(x_ref[...], w_ref[...], preferred_element_type=_f32)
    o_ref[0] = r[:, :DH]
    o_ref[1] = r[:, DH:]


def _mm_split(x, w, mb):
    m, k = x.shape
    return pl.pallas_call(
        _mm_split_body,
        grid=(m // mb,),
        in_specs=[pl.BlockSpec((mb, k), lambda i: (i, 0)),
                  pl.BlockSpec((k, D), lambda i: (0, 0))],
        out_specs=pl.BlockSpec((NC, mb, DH), lambda i: (0, i, 0)),
        out_shape=jax.ShapeDtypeStruct((NC, m, DH), _f32),
    )(x, w)


def _post_body(h_ref, agg_ref, deg_ref, ws_ref, wa_ref, o_ref):
    deg = deg_ref[0, :, 0:1] + deg_ref[1, :, 0:1]
    inv = 1.0 / jnp.maximum(deg, 1.0)
    agg = jnp.concatenate([agg_ref[0], agg_ref[1]], axis=1) * inv
    r = jnp.dot(h_ref[...], ws_ref[...], preferred_element_type=_f32)
    r = r + jnp.dot(agg, wa_ref[...], preferred_element_type=_f32)
    o_ref[...] = jnp.maximum(r, 0.0)


def _post(h, agg2, deg2, ws, wa, mb=1000):
    return pl.pallas_call(
        _post_body,
        grid=(N // mb,),
        in_specs=[pl.BlockSpec((mb, D), lambda i: (i, 0)),
                  pl.BlockSpec((NC, mb, DH), lambda i: (0, i, 0)),
                  pl.BlockSpec((NC, mb, DH), lambda i: (0, i, 0)),
                  pl.BlockSpec((D, D), lambda i: (0, 0)),
                  pl.BlockSpec((D, D), lambda i: (0, 0))],
        out_specs=pl.BlockSpec((mb, D), lambda i: (i, 0)),
        out_shape=jax.ShapeDtypeStruct((N, D), _f32),
    )(h, agg2, deg2, ws, wa)


def _mm_body(x_ref, w_ref, o_ref):
    o_ref[...] = jnp.dot(x_ref[...], w_ref[...], preferred_element_type=_f32)


def _mm(x, w, mb=1000):
    m, k = x.shape
    return pl.pallas_call(
        _mm_body,
        grid=(m // mb,),
        in_specs=[pl.BlockSpec((mb, k), lambda i: (i, 0)),
                  pl.BlockSpec((k, D), lambda i: (0, 0))],
        out_specs=pl.BlockSpec((mb, D), lambda i: (i, 0)),
        out_shape=jax.ShapeDtypeStruct((m, D), _f32),
    )(x, w)


def _edge_pass_body(a_hbm, e_hbm, src_hbm, dst_hbm, agg_out, deg_out,
                    idx_s, idx_d, idx_dd, msg, gbuf, zrow, ones_v,
                    agg_sh, sem):
    c = lax.axis_index("c")
    s = lax.axis_index("s")

    def _fill(i, _):
        r = i // 8
        g = i % 8
        zrow[r, pl.ds(g * 16, 16)] = jnp.zeros((16,), _f32)
        return 0
    lax.fori_loop(0, ZR * 8, _fill, 0)

    def _fill2(i, _):
        r = i // 8
        g = i % 8
        ones_v[r, pl.ds(g * 16, 16)] = jnp.ones((16,), _f32)
        return 0
    lax.fori_loop(0, EB_D * 8, _fill2, 0)

    row0 = s * ROWS_PT

    def _zero_acc():
        for k in range(ROWS_PT // ZR):
            pltpu.sync_copy(zrow, agg_sh.at[pl.ds(row0 + k * ZR, ZR)])

        @pl.when(s == NS - 1)
        def _():
            pltpu.sync_copy(zrow.at[pl.ds(0, TAILR)],
                            agg_sh.at[pl.ds(TAIL0, TAILR)])

    _zero_acc()
    plsc.subcore_barrier()

    coff = jnp.full((16,), c * N, jnp.int32)

    def _step(b, _):
        base = s * EPT + b * EB
        pltpu.sync_copy(src_hbm.at[pl.ds(base, EB)], idx_s)
        pltpu.sync_copy(dst_hbm.at[pl.ds(base, EB)], idx_d)
        for g in range(EB // 16):
            idx_s[pl.ds(g * 16, 16)] = idx_s[pl.ds(g * 16, 16)] + coff
        pltpu.sync_copy(e_hbm.at[pl.ds(c * E + base, EB)], msg)
        pltpu.async_copy(a_hbm.at[idx_s], gbuf, sem).wait()

        def _relu_row(r, _):
            for g in range(DH // 16):
                sl = pl.ds(g * 16, 16)
                msg[r, sl] = jnp.maximum(msg[r, sl] + gbuf[r, sl], 0.0)
            return 0
        lax.fori_loop(0, EB, _relu_row, 0)

        pltpu.sync_copy(msg, agg_sh.at[idx_d], add=True)
        return 0

    lax.fori_loop(0, STEPS, _step, 0)
    plsc.subcore_barrier()

    pltpu.sync_copy(agg_sh.at[pl.ds(row0, ROWS_PT)],
                    agg_out.at[pl.ds(c * N + row0, ROWS_PT)])

    @pl.when(s == NS - 1)
    def _():
        pltpu.sync_copy(agg_sh.at[pl.ds(TAIL0, TAILR)],
                        agg_out.at[pl.ds(c * N + TAIL0, TAILR)])
    plsc.subcore_barrier()

    _zero_acc()
    plsc.subcore_barrier()

    def _dstep(b, _):
        base = (c * NS + s) * EPT_D + b * EB_D
        pltpu.sync_copy(dst_hbm.at[pl.ds(base, EB_D)], idx_dd)
        pltpu.sync_copy(ones_v, agg_sh.at[idx_dd], add=True)
        return 0

    lax.fori_loop(0, STEPS_D, _dstep, 0)
    plsc.subcore_barrier()

    pltpu.sync_copy(agg_sh.at[pl.ds(row0, ROWS_PT)],
                    deg_out.at[pl.ds(c * N + row0, ROWS_PT)])

    @pl.when(s == NS - 1)
    def _():
        pltpu.sync_copy(agg_sh.at[pl.ds(TAIL0, TAILR)],
                        deg_out.at[pl.ds(c * N + TAIL0, TAILR)])


def _edge_pass(a2, e2, src, dst):
    mesh = plsc.VectorSubcoreMesh(core_axis_name="c", subcore_axis_name="s",
                                  num_cores=NC, num_subcores=NS)
    f = pl.kernel(
        _edge_pass_body,
        out_type=[jax.ShapeDtypeStruct((NC * N, DH), _f32),
                  jax.ShapeDtypeStruct((NC * N, DH), _f32)],
        mesh=mesh,
        scratch_types=[
            pltpu.VMEM((EB,), jnp.int32),
            pltpu.VMEM((EB,), jnp.int32),
            pltpu.VMEM((EB_D,), jnp.int32),
            pltpu.VMEM((EB, DH), _f32),
            pltpu.VMEM((EB, DH), _f32),
            pltpu.VMEM((ZR, DH), _f32),
            pltpu.VMEM((EB_D, DH), _f32),
            pltpu.VMEM_SHARED((N, DH), _f32),
            pltpu.SemaphoreType.DMA,
        ],
    )
    return f(a2, e2, src, dst)


def kernel(x, edge_index, edge_attr,
           W_msg0, W_edge0, W_self0, W_agg0,
           W_msg1, W_edge1, W_self1, W_agg1,
           W_out):
    src = edge_index[0]
    dst = edge_index[1]

    Wm = jnp.stack([W_msg0, W_msg1])
    We = jnp.stack([W_edge0, W_edge1])
    Ws = jnp.stack([W_self0, W_self1])
    Wa = jnp.stack([W_agg0, W_agg1])

    def layer(h, ws):
        wm, we, wself, wagg = ws
        e = _mm_split(edge_attr, we, 2000).reshape(NC * E, DH)
        a = _mm_split(h, wm, 1000).reshape(NC * N, DH)
        agg, deg2 = _edge_pass(a, e, src, dst)
        h2 = _post(h, agg.reshape(NC, N, DH), deg2.reshape(NC, N, DH),
                   wself, wagg)
        return h2, 0

    h_final, _ = lax.scan(layer, x, (Wm, We, Ws, Wa))
    return _mm(h_final, W_out)


# speedup vs baseline: 2.3035x; 1.3075x over previous
import jax
import jax.numpy as jnp
from jax import lax
from jax.experimental import pallas as pl
from jax.experimental.pallas import tpu as pltpu
from jax.experimental.pallas import tpu_sc as plsc

N = 10000
E = 160000
D = 256
DH = 128
NC = 2
NS = 16
EB = 40
EPT = E // NS
STEPS = EPT // EB
NGP = STEPS // 2
ROWS_PT = 624
TAIL0 = ROWS_PT * NS
TAILR = N - TAIL0
EB_D = 40
EPT_D = E // (NC * NS)
STEPS_D = EPT_D // EB_D
ZR = 48

_f32 = jnp.float32


def _mm_split_body(x_ref, w_ref, o_ref):
    r = jnp.dot(x_ref[...], w_ref[...], preferred_element_type=_f32)
    o_ref[0] = r[:, :DH]
    o_ref[1] = r[:, DH:]


def _mm_split(x, w, mb):
    m, k = x.shape
    return pl.pallas_call(
        _mm_split_body,
        grid=(m // mb,),
        in_specs=[pl.BlockSpec((mb, k), lambda i: (i, 0)),
                  pl.BlockSpec((k, D), lambda i: (0, 0))],
        out_specs=pl.BlockSpec((NC, mb, DH), lambda i: (0, i, 0)),
        out_shape=jax.ShapeDtypeStruct((NC, m, DH), _f32),
    )(x, w)


def _post_body(h_ref, agg_ref, deg_ref, ws_ref, wa_ref, o_ref):
    deg = deg_ref[0, :, 0:1] + deg_ref[1, :, 0:1]
    inv = 1.0 / jnp.maximum(deg, 1.0)
    agg = jnp.concatenate([agg_ref[0], agg_ref[1]], axis=1) * inv
    r = jnp.dot(h_ref[...], ws_ref[...], preferred_element_type=_f32)
    r = r + jnp.dot(agg, wa_ref[...], preferred_element_type=_f32)
    o_ref[...] = jnp.maximum(r, 0.0)


def _post(h, agg2, deg2, ws, wa, mb=1000):
    return pl.pallas_call(
        _post_body,
        grid=(N // mb,),
        in_specs=[pl.BlockSpec((mb, D), lambda i: (i, 0)),
                  pl.BlockSpec((NC, mb, DH), lambda i: (0, i, 0)),
                  pl.BlockSpec((NC, mb, DH), lambda i: (0, i, 0)),
                  pl.BlockSpec((D, D), lambda i: (0, 0)),
                  pl.BlockSpec((D, D), lambda i: (0, 0))],
        out_specs=pl.BlockSpec((mb, D), lambda i: (i, 0)),
        out_shape=jax.ShapeDtypeStruct((N, D), _f32),
    )(h, agg2, deg2, ws, wa)


def _mm_body(x_ref, w_ref, o_ref):
    o_ref[...] = jnp.dot(x_ref[...], w_ref[...], preferred_element_type=_f32)


def _mm(x, w, mb=1000):
    m, k = x.shape
    return pl.pallas_call(
        _mm_body,
        grid=(m // mb,),
        in_specs=[pl.BlockSpec((mb, k), lambda i: (i, 0)),
                  pl.BlockSpec((k, D), lambda i: (0, 0))],
        out_specs=pl.BlockSpec((mb, D), lambda i: (i, 0)),
        out_shape=jax.ShapeDtypeStruct((m, D), _f32),
    )(x, w)


def _edge_pass_body(a_hbm, e_hbm, src2_hbm, dst_hbm, agg_out, deg_out,
                    idx_s0, idx_s1, idx_d0, idx_d1, idx_sc0, idx_sc1, idx_dd,
                    eb0, eb1, gb0, gb1, sb0, sb1, zrow, agg_sh,
                    sl0, sl1, sg0, sg1, ssc0, ssc1):
    c = lax.axis_index("c")
    s = lax.axis_index("s")
    slot = ((idx_s0, idx_d0, idx_sc0, eb0, gb0, sb0, sl0, sg0, ssc0),
            (idx_s1, idx_d1, idx_sc1, eb1, gb1, sb1, sl1, sg1, ssc1))

    def _fill(i, _):
        r = i // 8
        g = i % 8
        zrow[r, pl.ds(g * 16, 16)] = jnp.zeros((16,), _f32)
        return 0
    lax.fori_loop(0, ZR * 8, _fill, 0)

    row0 = s * ROWS_PT

    def _zero_acc():
        for k in range(ROWS_PT // ZR):
            pltpu.sync_copy(zrow, agg_sh.at[pl.ds(row0 + k * ZR, ZR)])

        @pl.when(s == NS - 1)
        def _():
            pltpu.sync_copy(zrow.at[pl.ds(0, TAILR)],
                            agg_sh.at[pl.ds(TAIL0, TAILR)])

    _zero_acc()
    plsc.subcore_barrier()

    def issue_loads(p, b):
        ids, idd, _, ebf, _, _, sl, _, _ = slot[p]
        base = s * EPT + b * EB
        pltpu.async_copy(src2_hbm.at[pl.ds(c * E + base, EB)], ids, sl)
        pltpu.async_copy(dst_hbm.at[pl.ds(base, EB)], idd, sl)
        pltpu.async_copy(e_hbm.at[pl.ds(c * E + base, EB)], ebf, sl)

    def wait_loads(p):
        ids, idd, _, ebf, _, _, sl, _, _ = slot[p]
        pltpu.make_async_copy(src2_hbm.at[pl.ds(0, EB)], ids, sl).wait()
        pltpu.make_async_copy(dst_hbm.at[pl.ds(0, EB)], idd, sl).wait()
        pltpu.make_async_copy(e_hbm.at[pl.ds(0, EB)], ebf, sl).wait()

    def issue_gather(p):
        ids, _, _, _, gbf, _, _, sg, _ = slot[p]
        pltpu.async_copy(a_hbm.at[ids], gbf, sg)

    issue_loads(0, 0)
    issue_loads(1, 1)
    wait_loads(0)
    issue_gather(0)

    def _pair(g, _):
        for p in (0, 1):
            q = 1 - p
            b = 2 * g + p
            ids, idd, isc, ebf, gbf, sbf, sl, sg, ssc = slot[p]
            pltpu.make_async_copy(a_hbm.at[ids], gbf, sg).wait()

            @pl.when(g >= 1)
            def _():
                pltpu.make_async_copy(sbf, agg_sh.at[isc], ssc).wait()

            def _relu_row(r, _):
                for gg in range(DH // 16):
                    d = pl.ds(gg * 16, 16)
                    sbf[r, d] = jnp.maximum(ebf[r, d] + gbf[r, d], 0.0)
                return 0
            lax.fori_loop(0, EB, _relu_row, 0)

            for off in (0, 16, EB - 16):
                isc[pl.ds(off, 16)] = idd[pl.ds(off, 16)]
            pltpu.async_copy(sbf, agg_sh.at[isc], ssc, add=True)

            @pl.when(g < NGP - 1)
            def _():
                issue_loads(p, b + 2)

            if p == 0:
                wait_loads(q)
                issue_gather(q)
            else:
                @pl.when(g < NGP - 1)
                def _():
                    wait_loads(q)
                    issue_gather(q)
        return 0

    lax.fori_loop(0, NGP, _pair, 0)

    for p in (0, 1):
        _, _, isc, _, _, sbf, _, _, ssc = slot[p]
        pltpu.make_async_copy(sbf, agg_sh.at[isc], ssc).wait()
    plsc.subcore_barrier()

    pltpu.sync_copy(agg_sh.at[pl.ds(row0, ROWS_PT)],
                    agg_out.at[pl.ds(c * N + row0, ROWS_PT)])

    @pl.when(s == NS - 1)
    def _():
        pltpu.sync_copy(agg_sh.at[pl.ds(TAIL0, TAILR)],
                        agg_out.at[pl.ds(c * N + TAIL0, TAILR)])
    plsc.subcore_barrier()

    _zero_acc()

    def _fill2(i, _):
        r = i // 8
        g = i % 8
        sb0[r, pl.ds(g * 16, 16)] = jnp.ones((16,), _f32)
        return 0
    lax.fori_loop(0, EB_D * 8, _fill2, 0)
    plsc.subcore_barrier()

    def _dstep(b, _):
        base = (c * NS + s) * EPT_D + b * EB_D
        pltpu.sync_copy(dst_hbm.at[pl.ds(base, EB_D)], idx_dd)
        pltpu.sync_copy(sb0, agg_sh.at[idx_dd], add=True)
        return 0

    lax.fori_loop(0, STEPS_D, _dstep, 0)
    plsc.subcore_barrier()

    pltpu.sync_copy(agg_sh.at[pl.ds(row0, ROWS_PT)],
                    deg_out.at[pl.ds(c * N + row0, ROWS_PT)])

    @pl.when(s == NS - 1)
    def _():
        pltpu.sync_copy(agg_sh.at[pl.ds(TAIL0, TAILR)],
                        deg_out.at[pl.ds(c * N + TAIL0, TAILR)])


def _edge_pass(a2, e2, src2, dst):
    mesh = plsc.VectorSubcoreMesh(core_axis_name="c", subcore_axis_name="s",
                                  num_cores=NC, num_subcores=NS)
    f = pl.kernel(
        _edge_pass_body,
        out_type=[jax.ShapeDtypeStruct((NC * N, DH), _f32),
                  jax.ShapeDtypeStruct((NC * N, DH), _f32)],
        mesh=mesh,
        scratch_types=[
            pltpu.VMEM((EB,), jnp.int32),
            pltpu.VMEM((EB,), jnp.int32),
            pltpu.VMEM((EB,), jnp.int32),
            pltpu.VMEM((EB,), jnp.int32),
            pltpu.VMEM((EB,), jnp.int32),
            pltpu.VMEM((EB,), jnp.int32),
            pltpu.VMEM((EB_D,), jnp.int32),
            pltpu.VMEM((EB, DH), _f32),
            pltpu.VMEM((EB, DH), _f32),
            pltpu.VMEM((EB, DH), _f32),
            pltpu.VMEM((EB, DH), _f32),
            pltpu.VMEM((EB, DH), _f32),
            pltpu.VMEM((EB, DH), _f32),
            pltpu.VMEM((ZR, DH), _f32),
            pltpu.VMEM_SHARED((N, DH), _f32),
            pltpu.SemaphoreType.DMA,
            pltpu.SemaphoreType.DMA,
            pltpu.SemaphoreType.DMA,
            pltpu.SemaphoreType.DMA,
            pltpu.SemaphoreType.DMA,
            pltpu.SemaphoreType.DMA,
        ],
    )
    return f(a2, e2, src2, dst)


def kernel(x, edge_index, edge_attr,
           W_msg0, W_edge0, W_self0, W_agg0,
           W_msg1, W_edge1, W_self1, W_agg1,
           W_out):
    src = edge_index[0]
    dst = edge_index[1]
    src2 = jnp.concatenate([src, src + N])

    Wm = jnp.stack([W_msg0, W_msg1])
    We = jnp.stack([W_edge0, W_edge1])
    Ws = jnp.stack([W_self0, W_self1])
    Wa = jnp.stack([W_agg0, W_agg1])

    def layer(h, ws):
        wm, we, wself, wagg = ws
        e = _mm_split(edge_attr, we, 2000).reshape(NC * E, DH)
        a = _mm_split(h, wm, 1000).reshape(NC * N, DH)
        agg, deg2 = _edge_pass(a, e, src2, dst)
        h2 = _post(h, agg.reshape(NC, N, DH), deg2.reshape(NC, N, DH),
                   wself, wagg)
        return h2, 0

    h_final, _ = lax.scan(layer, x, (Wm, We, Ws, Wa))
    return _mm(h_final, W_out)


# speedup vs baseline: 2.5289x; 1.0979x over previous
import jax
import jax.numpy as jnp
from jax import lax
from jax.experimental import pallas as pl
from jax.experimental.pallas import tpu as pltpu
from jax.experimental.pallas import tpu_sc as plsc

N = 10000
E = 160000
D = 256
DH = 128
NC = 2
NS = 16
EB = 40
EPT = E // NS
STEPS = EPT // EB
NGP = STEPS // 2
ROWS_PT = 624
TAIL0 = ROWS_PT * NS
TAILR = N - TAIL0
EB_D = 40
EPT_D = E // (NC * NS)
STEPS_D = EPT_D // EB_D
NGP_D = (STEPS_D - 1) // 2
ZR = 48

_f32 = jnp.float32


def _mm_split_body(x_ref, w_ref, o_ref):
    r = jnp.dot(x_ref[...], w_ref[...], preferred_element_type=_f32)
    o_ref[0] = r[:, :DH]
    o_ref[1] = r[:, DH:]


def _mm_split(x, w, mb):
    m, k = x.shape
    return pl.pallas_call(
        _mm_split_body,
        grid=(m // mb,),
        in_specs=[pl.BlockSpec((mb, k), lambda i: (i, 0)),
                  pl.BlockSpec((k, D), lambda i: (0, 0))],
        out_specs=pl.BlockSpec((NC, mb, DH), lambda i: (0, i, 0)),
        out_shape=jax.ShapeDtypeStruct((NC, m, DH), _f32),
    )(x, w)


def _post_body(h_ref, agg_ref, deg_ref, ws_ref, wa_ref, o_ref):
    deg = deg_ref[0, :, 0:1] + deg_ref[1, :, 0:1]
    inv = 1.0 / jnp.maximum(deg, 1.0)
    agg = jnp.concatenate([agg_ref[0], agg_ref[1]], axis=1) * inv
    r = jnp.dot(h_ref[...], ws_ref[...], preferred_element_type=_f32)
    r = r + jnp.dot(agg, wa_ref[...], preferred_element_type=_f32)
    o_ref[...] = jnp.maximum(r, 0.0)


def _post(h, agg2, deg2, ws, wa, mb=1000):
    return pl.pallas_call(
        _post_body,
        grid=(N // mb,),
        in_specs=[pl.BlockSpec((mb, D), lambda i: (i, 0)),
                  pl.BlockSpec((NC, mb, DH), lambda i: (0, i, 0)),
                  pl.BlockSpec((NC, mb, DH), lambda i: (0, i, 0)),
                  pl.BlockSpec((D, D), lambda i: (0, 0)),
                  pl.BlockSpec((D, D), lambda i: (0, 0))],
        out_specs=pl.BlockSpec((mb, D), lambda i: (i, 0)),
        out_shape=jax.ShapeDtypeStruct((N, D), _f32),
    )(h, agg2, deg2, ws, wa)


def _mm_body(x_ref, w_ref, o_ref):
    o_ref[...] = jnp.dot(x_ref[...], w_ref[...], preferred_element_type=_f32)


def _mm(x, w, mb=1000):
    m, k = x.shape
    return pl.pallas_call(
        _mm_body,
        grid=(m // mb,),
        in_specs=[pl.BlockSpec((mb, k), lambda i: (i, 0)),
                  pl.BlockSpec((k, D), lambda i: (0, 0))],
        out_specs=pl.BlockSpec((mb, D), lambda i: (i, 0)),
        out_shape=jax.ShapeDtypeStruct((m, D), _f32),
    )(x, w)


def _edge_pass_body(a_hbm, e_hbm, src2_hbm, dst_hbm, agg_out, deg_out,
                    idx_s0, idx_s1, idx_d0, idx_d1, idx_sc0, idx_sc1,
                    eb0, eb1, gb0, gb1, sb0, sb1, zrow, agg_sh,
                    sl0, sl1, sg0, sg1, ssc0, ssc1):
    c = lax.axis_index("c")
    s = lax.axis_index("s")
    slot = ((idx_s0, idx_d0, idx_sc0, eb0, gb0, sb0, sl0, sg0, ssc0),
            (idx_s1, idx_d1, idx_sc1, eb1, gb1, sb1, sl1, sg1, ssc1))

    def _fill(i, _):
        r = i // 8
        g = i % 8
        zrow[r, pl.ds(g * 16, 16)] = jnp.zeros((16,), _f32)
        return 0
    lax.fori_loop(0, ZR * 8, _fill, 0)

    row0 = s * ROWS_PT

    def _zero_acc():
        for k in range(ROWS_PT // ZR):
            pltpu.sync_copy(zrow, agg_sh.at[pl.ds(row0 + k * ZR, ZR)])

        @pl.when(s == NS - 1)
        def _():
            pltpu.sync_copy(zrow.at[pl.ds(0, TAILR)],
                            agg_sh.at[pl.ds(TAIL0, TAILR)])

    _zero_acc()
    plsc.subcore_barrier()

    def issue_loads(p, b):
        ids, idd, _, ebf, _, _, sl, _, _ = slot[p]
        base = s * EPT + b * EB
        pltpu.async_copy(src2_hbm.at[pl.ds(c * E + base, EB)], ids, sl)
        pltpu.async_copy(dst_hbm.at[pl.ds(base, EB)], idd, sl)
        pltpu.async_copy(e_hbm.at[pl.ds(c * E + base, EB)], ebf, sl)

    def wait_loads(p):
        ids, idd, _, ebf, _, _, sl, _, _ = slot[p]
        pltpu.make_async_copy(src2_hbm.at[pl.ds(0, EB)], ids, sl).wait()
        pltpu.make_async_copy(dst_hbm.at[pl.ds(0, EB)], idd, sl).wait()
        pltpu.make_async_copy(e_hbm.at[pl.ds(0, EB)], ebf, sl).wait()

    def issue_gather(p):
        ids, _, _, _, gbf, _, _, sg, _ = slot[p]
        pltpu.async_copy(a_hbm.at[ids], gbf, sg)

    issue_loads(0, 0)
    issue_loads(1, 1)
    wait_loads(0)
    issue_gather(0)

    def _pair(g, _):
        for p in (0, 1):
            q = 1 - p
            b = 2 * g + p
            ids, idd, isc, ebf, gbf, sbf, sl, sg, ssc = slot[p]
            pltpu.make_async_copy(a_hbm.at[ids], gbf, sg).wait()

            @pl.when(g >= 1)
            def _():
                pltpu.make_async_copy(sbf, agg_sh.at[isc], ssc).wait()

            def _relu_row(r, _):
                for gg in range(DH // 16):
                    d = pl.ds(gg * 16, 16)
                    sbf[r, d] = jnp.maximum(ebf[r, d] + gbf[r, d], 0.0)
                return 0
            lax.fori_loop(0, EB, _relu_row, 0)

            for off in (0, 16, EB - 16):
                isc[pl.ds(off, 16)] = idd[pl.ds(off, 16)]
            pltpu.async_copy(sbf, agg_sh.at[isc], ssc, add=True)

            @pl.when(g < NGP - 1)
            def _():
                issue_loads(p, b + 2)

            if p == 0:
                wait_loads(q)
                issue_gather(q)
            else:
                @pl.when(g < NGP - 1)
                def _():
                    wait_loads(q)
                    issue_gather(q)
        return 0

    lax.fori_loop(0, NGP, _pair, 0)

    for p in (0, 1):
        _, _, isc, _, _, sbf, _, _, ssc = slot[p]
        pltpu.make_async_copy(sbf, agg_sh.at[isc], ssc).wait()
    plsc.subcore_barrier()

    pltpu.sync_copy(agg_sh.at[pl.ds(row0, ROWS_PT)],
                    agg_out.at[pl.ds(c * N + row0, ROWS_PT)])

    @pl.when(s == NS - 1)
    def _():
        pltpu.sync_copy(agg_sh.at[pl.ds(TAIL0, TAILR)],
                        agg_out.at[pl.ds(c * N + TAIL0, TAILR)])
    plsc.subcore_barrier()

    if True:
        _zero_acc()

        def _fill2(i, _):
            r = i // 8
            g = i % 8
            sb0[r, pl.ds(g * 16, 16)] = jnp.ones((16,), _f32)
            return 0
        lax.fori_loop(0, EB_D * 8, _fill2, 0)
    plsc.subcore_barrier()

    if True:
        def dissue(p, b):
            base = (c * NS + s) * EPT_D + b * EB_D
            pltpu.async_copy(dst_hbm.at[pl.ds(base, EB_D)],
                             slot[p][0], slot[p][6])

        def dwait(p):
            pltpu.make_async_copy(dst_hbm.at[pl.ds(0, EB_D)],
                                  slot[p][0], slot[p][6]).wait()

        def dcopy_idx(p):
            for off in (0, 16, EB_D - 16):
                slot[p][2][pl.ds(off, 16)] = slot[p][0][pl.ds(off, 16)]

        dissue(0, 0)
        dissue(1, 1)

        def _dpair(g, _):
            for p in (0, 1):
                b = 2 * g + p
                isc, ssc = slot[p][2], slot[p][8]
                dwait(p)

                @pl.when(g >= 1)
                def _():
                    pltpu.make_async_copy(sb0, agg_sh.at[isc], ssc).wait()
                dcopy_idx(p)
                pltpu.async_copy(sb0, agg_sh.at[isc], ssc, add=True)
                if p == 0:
                    dissue(p, b + 2)
                else:
                    @pl.when(g < NGP_D - 1)
                    def _():
                        dissue(p, b + 2)
            return 0

        lax.fori_loop(0, NGP_D, _dpair, 0)

        dwait(0)
        pltpu.make_async_copy(sb0, agg_sh.at[slot[0][2]], slot[0][8]).wait()
        dcopy_idx(0)
        pltpu.async_copy(sb0, agg_sh.at[slot[0][2]], slot[0][8], add=True)
        pltpu.make_async_copy(sb0, agg_sh.at[slot[0][2]], slot[0][8]).wait()
        pltpu.make_async_copy(sb0, agg_sh.at[slot[1][2]], slot[1][8]).wait()
    plsc.subcore_barrier()

    pltpu.sync_copy(agg_sh.at[pl.ds(row0, ROWS_PT)],
                    deg_out.at[pl.ds(c * N + row0, ROWS_PT)])

    @pl.when(s == NS - 1)
    def _():
        pltpu.sync_copy(agg_sh.at[pl.ds(TAIL0, TAILR)],
                        deg_out.at[pl.ds(c * N + TAIL0, TAILR)])


def _edge_pass(a2, e2, src2, dst):
    mesh = plsc.VectorSubcoreMesh(core_axis_name="c", subcore_axis_name="s",
                                  num_cores=NC, num_subcores=NS)
    f = pl.kernel(
        _edge_pass_body,
        out_type=[jax.ShapeDtypeStruct((NC * N, DH), _f32),
                  jax.ShapeDtypeStruct((NC * N, DH), _f32)],
        mesh=mesh,
        scratch_types=[
            pltpu.VMEM((EB,), jnp.int32),
            pltpu.VMEM((EB,), jnp.int32),
            pltpu.VMEM((EB,), jnp.int32),
            pltpu.VMEM((EB,), jnp.int32),
            pltpu.VMEM((EB,), jnp.int32),
            pltpu.VMEM((EB,), jnp.int32),
            pltpu.VMEM((EB, DH), _f32),
            pltpu.VMEM((EB, DH), _f32),
            pltpu.VMEM((EB, DH), _f32),
            pltpu.VMEM((EB, DH), _f32),
            pltpu.VMEM((EB, DH), _f32),
            pltpu.VMEM((EB, DH), _f32),
            pltpu.VMEM((ZR, DH), _f32),
            pltpu.VMEM_SHARED((N, DH), _f32),
            pltpu.SemaphoreType.DMA,
            pltpu.SemaphoreType.DMA,
            pltpu.SemaphoreType.DMA,
            pltpu.SemaphoreType.DMA,
            pltpu.SemaphoreType.DMA,
            pltpu.SemaphoreType.DMA,
        ],
    )
    return f(a2, e2, src2, dst)


def kernel(x, edge_index, edge_attr,
           W_msg0, W_edge0, W_self0, W_agg0,
           W_msg1, W_edge1, W_self1, W_agg1,
           W_out):
    src = edge_index[0]
    dst = edge_index[1]
    src2 = jnp.concatenate([src, src + N])

    Wm = jnp.stack([W_msg0, W_msg1])
    We = jnp.stack([W_edge0, W_edge1])
    Ws = jnp.stack([W_self0, W_self1])
    Wa = jnp.stack([W_agg0, W_agg1])

    def layer(h, ws):
        wm, we, wself, wagg = ws
        e = _mm_split(edge_attr, we, 2000).reshape(NC * E, DH)
        a = _mm_split(h, wm, 1000).reshape(NC * N, DH)
        agg, deg2 = _edge_pass(a, e, src2, dst)
        h2 = _post(h, agg.reshape(NC, N, DH), deg2.reshape(NC, N, DH),
                   wself, wagg)
        return h2, 0

    h_final, _ = lax.scan(layer, x, (Wm, We, Ws, Wa))
    return _mm(h_final, W_out)


# speedup vs baseline: 2.8408x; 1.1233x over previous
import jax
import jax.numpy as jnp
from jax import lax
from jax.experimental import pallas as pl
from jax.experimental.pallas import tpu as pltpu
from jax.experimental.pallas import tpu_sc as plsc

N = 10000
E = 160000
D = 256
DH = 128
NC = 2
NS = 16
EB = 80
EPT = E // NS
STEPS = EPT // EB
NGP = (STEPS - 1) // 2
ROWS_PT = 624
TAIL0 = ROWS_PT * NS
TAILR = N - TAIL0
EB_D = 40
EPT_D = E // (NC * NS)
STEPS_D = EPT_D // EB_D
NGP_D = (STEPS_D - 1) // 2
ZR = 48

_f32 = jnp.float32


def _mm_split_body(x_ref, w_ref, o_ref):
    r = jnp.dot(x_ref[...], w_ref[...], preferred_element_type=_f32)
    o_ref[0] = r[:, :DH]
    o_ref[1] = r[:, DH:]


def _mm_split(x, w, mb):
    m, k = x.shape
    return pl.pallas_call(
        _mm_split_body,
        grid=(m // mb,),
        in_specs=[pl.BlockSpec((mb, k), lambda i: (i, 0)),
                  pl.BlockSpec((k, D), lambda i: (0, 0))],
        out_specs=pl.BlockSpec((NC, mb, DH), lambda i: (0, i, 0)),
        out_shape=jax.ShapeDtypeStruct((NC, m, DH), _f32),
    )(x, w)


def _post_body(h_ref, agg_ref, deg_ref, ws_ref, wa_ref, o_ref):
    deg = deg_ref[0, :, 0:1] + deg_ref[1, :, 0:1]
    inv = 1.0 / jnp.maximum(deg, 1.0)
    agg = jnp.concatenate([agg_ref[0], agg_ref[1]], axis=1) * inv
    r = jnp.dot(h_ref[...], ws_ref[...], preferred_element_type=_f32)
    r = r + jnp.dot(agg, wa_ref[...], preferred_element_type=_f32)
    o_ref[...] = jnp.maximum(r, 0.0)


def _post(h, agg2, deg2, ws, wa, mb=1000):
    return pl.pallas_call(
        _post_body,
        grid=(N // mb,),
        in_specs=[pl.BlockSpec((mb, D), lambda i: (i, 0)),
                  pl.BlockSpec((NC, mb, DH), lambda i: (0, i, 0)),
                  pl.BlockSpec((NC, mb, DH), lambda i: (0, i, 0)),
                  pl.BlockSpec((D, D), lambda i: (0, 0)),
                  pl.BlockSpec((D, D), lambda i: (0, 0))],
        out_specs=pl.BlockSpec((mb, D), lambda i: (i, 0)),
        out_shape=jax.ShapeDtypeStruct((N, D), _f32),
    )(h, agg2, deg2, ws, wa)


def _mm_body(x_ref, w_ref, o_ref):
    o_ref[...] = jnp.dot(x_ref[...], w_ref[...], preferred_element_type=_f32)


def _mm(x, w, mb=1000):
    m, k = x.shape
    return pl.pallas_call(
        _mm_body,
        grid=(m // mb,),
        in_specs=[pl.BlockSpec((mb, k), lambda i: (i, 0)),
                  pl.BlockSpec((k, D), lambda i: (0, 0))],
        out_specs=pl.BlockSpec((mb, D), lambda i: (i, 0)),
        out_shape=jax.ShapeDtypeStruct((m, D), _f32),
    )(x, w)


def _edge_pass_body(a_hbm, e_hbm, src2_hbm, dst_hbm, agg_out, deg_out,
                    idx_s0, idx_s1, idx_d0, idx_d1, idx_sc0, idx_sc1,
                    idx_dd0, idx_dd1, iscd0, iscd1,
                    eb0, eb1, sb0, sb1, zrow, agg_sh,
                    sl0, sl1, sg0, sg1, ssc0, ssc1):
    c = lax.axis_index("c")
    s = lax.axis_index("s")
    slot = ((idx_s0, idx_d0, idx_sc0, eb0, sb0, sl0, sg0, ssc0),
            (idx_s1, idx_d1, idx_sc1, eb1, sb1, sl1, sg1, ssc1))
    dslot = ((idx_dd0, iscd0, sl0, ssc0), (idx_dd1, iscd1, sl1, ssc1))

    def _fill(i, _):
        r = i // 8
        g = i % 8
        zrow[r, pl.ds(g * 16, 16)] = jnp.zeros((16,), _f32)
        return 0
    lax.fori_loop(0, ZR * 8, _fill, 0)

    row0 = s * ROWS_PT

    def _zero_acc():
        for k in range(ROWS_PT // ZR):
            pltpu.sync_copy(zrow, agg_sh.at[pl.ds(row0 + k * ZR, ZR)])

        @pl.when(s == NS - 1)
        def _():
            pltpu.sync_copy(zrow.at[pl.ds(0, TAILR)],
                            agg_sh.at[pl.ds(TAIL0, TAILR)])

    _zero_acc()
    plsc.subcore_barrier()

    def issue_loads(p, b):
        ids, idd, _, ebf, _, sl, _, _ = slot[p]
        base = s * EPT + b * EB
        pltpu.async_copy(src2_hbm.at[pl.ds(c * E + base, EB)], ids, sl)
        pltpu.async_copy(dst_hbm.at[pl.ds(base, EB)], idd, sl)
        pltpu.async_copy(e_hbm.at[pl.ds(c * E + base, EB)], ebf, sl)

    def wait_loads(p):
        ids, idd, _, ebf, _, sl, _, _ = slot[p]
        pltpu.make_async_copy(src2_hbm.at[pl.ds(0, EB)], ids, sl).wait()
        pltpu.make_async_copy(dst_hbm.at[pl.ds(0, EB)], idd, sl).wait()
        pltpu.make_async_copy(e_hbm.at[pl.ds(0, EB)], ebf, sl).wait()

    def issue_gather(p):
        ids, _, _, _, sbf, _, sg, _ = slot[p]
        pltpu.async_copy(a_hbm.at[ids], sbf, sg)

    def wait_gather(p):
        ids, _, _, _, sbf, _, sg, _ = slot[p]
        pltpu.make_async_copy(a_hbm.at[ids], sbf, sg).wait()

    def wait_scatter(p):
        _, _, isc, _, sbf, _, _, ssc = slot[p]
        pltpu.make_async_copy(sbf, agg_sh.at[isc], ssc).wait()

    def do_step(p):
        ids, idd, isc, ebf, sbf, sl, sg, ssc = slot[p]
        wait_gather(p)

        def _relu_row(r, _):
            for gg in range(DH // 16):
                d = pl.ds(gg * 16, 16)
                sbf[r, d] = jnp.maximum(ebf[r, d] + sbf[r, d], 0.0)
            return 0
        lax.fori_loop(0, EB, _relu_row, 0)
        for off in range(0, EB, 16):
            isc[pl.ds(off, 16)] = idd[pl.ds(off, 16)]
        pltpu.async_copy(sbf, agg_sh.at[isc], ssc, add=True)

    issue_loads(0, 0)
    issue_loads(1, 1)
    wait_loads(0)
    issue_gather(0)

    def _pair(g, _):
        for p in (0, 1):
            q = 1 - p
            b = 2 * g + p
            do_step(p)
            if p == 0:
                issue_loads(p, b + 2)
            else:
                @pl.when(g < NGP - 1)
                def _():
                    issue_loads(p, b + 2)
            wait_loads(q)
            if p == 0:
                @pl.when(g >= 1)
                def _():
                    wait_scatter(q)
            else:
                wait_scatter(q)
            issue_gather(q)
        return 0

    lax.fori_loop(0, NGP, _pair, 0)

    do_step(0)
    wait_scatter(0)
    wait_scatter(1)
    plsc.subcore_barrier()

    pltpu.sync_copy(agg_sh.at[pl.ds(row0, ROWS_PT)],
                    agg_out.at[pl.ds(c * N + row0, ROWS_PT)])

    @pl.when(s == NS - 1)
    def _():
        pltpu.sync_copy(agg_sh.at[pl.ds(TAIL0, TAILR)],
                        agg_out.at[pl.ds(c * N + TAIL0, TAILR)])
    plsc.subcore_barrier()

    if True:
        _zero_acc()

        def _fill2(i, _):
            r = i // 8
            g = i % 8
            sb0[r, pl.ds(g * 16, 16)] = jnp.ones((16,), _f32)
            return 0
        lax.fori_loop(0, EB_D * 8, _fill2, 0)
    plsc.subcore_barrier()

    if True:
        ones_src = sb0.at[pl.ds(0, EB_D)]

        def dissue(p, b):
            idd, _, sl, _ = dslot[p]
            base = (c * NS + s) * EPT_D + b * EB_D
            pltpu.async_copy(dst_hbm.at[pl.ds(base, EB_D)], idd, sl)

        def dwait(p):
            idd, _, sl, _ = dslot[p]
            pltpu.make_async_copy(dst_hbm.at[pl.ds(0, EB_D)], idd, sl).wait()

        def dcopy_idx(p):
            idd, isc, _, _ = dslot[p]
            for off in (0, 16, EB_D - 16):
                isc[pl.ds(off, 16)] = idd[pl.ds(off, 16)]

        dissue(0, 0)
        dissue(1, 1)

        def _dpair(g, _):
            for p in (0, 1):
                b = 2 * g + p
                _, isc, _, ssc = dslot[p]
                dwait(p)

                @pl.when(g >= 1)
                def _():
                    pltpu.make_async_copy(ones_src, agg_sh.at[isc], ssc).wait()
                dcopy_idx(p)
                pltpu.async_copy(ones_src, agg_sh.at[isc], ssc, add=True)
                if p == 0:
                    dissue(p, b + 2)
                else:
                    @pl.when(g < NGP_D - 1)
                    def _():
                        dissue(p, b + 2)
            return 0

        lax.fori_loop(0, NGP_D, _dpair, 0)

        dwait(0)
        pltpu.make_async_copy(ones_src, agg_sh.at[dslot[0][1]], dslot[0][3]).wait()
        dcopy_idx(0)
        pltpu.async_copy(ones_src, agg_sh.at[dslot[0][1]], dslot[0][3], add=True)
        pltpu.make_async_copy(ones_src, agg_sh.at[dslot[0][1]], dslot[0][3]).wait()
        pltpu.make_async_copy(ones_src, agg_sh.at[dslot[1][1]], dslot[1][3]).wait()
    plsc.subcore_barrier()

    pltpu.sync_copy(agg_sh.at[pl.ds(row0, ROWS_PT)],
                    deg_out.at[pl.ds(c * N + row0, ROWS_PT)])

    @pl.when(s == NS - 1)
    def _():
        pltpu.sync_copy(agg_sh.at[pl.ds(TAIL0, TAILR)],
                        deg_out.at[pl.ds(c * N + TAIL0, TAILR)])


def _edge_pass(a2, e2, src2, dst):
    mesh = plsc.VectorSubcoreMesh(core_axis_name="c", subcore_axis_name="s",
                                  num_cores=NC, num_subcores=NS)
    f = pl.kernel(
        _edge_pass_body,
        out_type=[jax.ShapeDtypeStruct((NC * N, DH), _f32),
                  jax.ShapeDtypeStruct((NC * N, DH), _f32)],
        mesh=mesh,
        scratch_types=[
            pltpu.VMEM((EB,), jnp.int32),
            pltpu.VMEM((EB,), jnp.int32),
            pltpu.VMEM((EB,), jnp.int32),
            pltpu.VMEM((EB,), jnp.int32),
            pltpu.VMEM((EB,), jnp.int32),
            pltpu.VMEM((EB,), jnp.int32),
            pltpu.VMEM((EB_D,), jnp.int32),
            pltpu.VMEM((EB_D,), jnp.int32),
            pltpu.VMEM((EB_D,), jnp.int32),
            pltpu.VMEM((EB_D,), jnp.int32),
            pltpu.VMEM((EB, DH), _f32),
            pltpu.VMEM((EB, DH), _f32),
            pltpu.VMEM((EB, DH), _f32),
            pltpu.VMEM((EB, DH), _f32),
            pltpu.VMEM((ZR, DH), _f32),
            pltpu.VMEM_SHARED((N, DH), _f32),
            pltpu.SemaphoreType.DMA,
            pltpu.SemaphoreType.DMA,
            pltpu.SemaphoreType.DMA,
            pltpu.SemaphoreType.DMA,
            pltpu.SemaphoreType.DMA,
            pltpu.SemaphoreType.DMA,
        ],
    )
    return f(a2, e2, src2, dst)


def kernel(x, edge_index, edge_attr,
           W_msg0, W_edge0, W_self0, W_agg0,
           W_msg1, W_edge1, W_self1, W_agg1,
           W_out):
    src = edge_index[0]
    dst = edge_index[1]
    src2 = jnp.concatenate([src, src + N])

    Wm = jnp.stack([W_msg0, W_msg1])
    We = jnp.stack([W_edge0, W_edge1])
    Ws = jnp.stack([W_self0, W_self1])
    Wa = jnp.stack([W_agg0, W_agg1])

    def layer(h, ws):
        wm, we, wself, wagg = ws
        e = _mm_split(edge_attr, we, 2000).reshape(NC * E, DH)
        a = _mm_split(h, wm, 1000).reshape(NC * N, DH)
        agg, deg2 = _edge_pass(a, e, src2, dst)
        h2 = _post(h, agg.reshape(NC, N, DH), deg2.reshape(NC, N, DH),
                   wself, wagg)
        return h2, 0

    h_final, _ = lax.scan(layer, x, (Wm, We, Ws, Wa))
    return _mm(h_final, W_out)
